# Initial kernel scaffold; baseline (speedup 1.0000x reference)
import jax
import jax.numpy as jnp
from jax.experimental import pallas as pl


def kernel(x, tables, W, b, gamma, beta):
    raise NotImplementedError("write your pallas kernel here")


# speedup vs baseline: 15.9883x; 15.9883x over previous
import functools

import jax
import jax.numpy as jnp
from jax import lax
from jax.experimental import pallas as pl
from jax.experimental.pallas import tpu as pltpu
from jax.experimental.pallas import tpu_sc as plsc

N_FIELDS = 26
CARD = 10000
EMB_D = 101
D_MODEL = 128
BATCH = 16384
EPS = 1e-5

TOTAL = N_FIELDS * BATCH
BM = 1000

NC = 2
NS = 16
NW = NC * NS
PER_W = TOTAL // NW
CH = 512
N_CHUNKS = PER_W // CH
LOG2_BATCH = 14


def _tc_project_body(tbl_ref, w_ref, b_ref, g_ref, be_ref, out_ref):
    emb = tbl_ref[0]
    w = w_ref[0]
    prj = jnp.dot(emb, w, preferred_element_type=jnp.float32)
    prj = prj + b_ref[0][0][None, :]
    mean = jnp.mean(prj, axis=-1, keepdims=True)
    cent = prj - mean
    var = jnp.mean(cent * cent, axis=-1, keepdims=True)
    inv = lax.rsqrt(var + EPS)
    out_ref[0] = cent * inv * g_ref[0][0][None, :] + be_ref[0][0][None, :]


def _project_tables(tables, W, b, gamma, beta):
    b3 = b[:, None, :]
    g3 = gamma[:, None, :]
    be3 = beta[:, None, :]
    return pl.pallas_call(
        _tc_project_body,
        grid=(N_FIELDS, CARD // BM),
        in_specs=[
            pl.BlockSpec((1, BM, EMB_D), lambda f, m: (f, m, 0)),
            pl.BlockSpec((1, EMB_D, D_MODEL), lambda f, m: (f, 0, 0)),
            pl.BlockSpec((1, 1, D_MODEL), lambda f, m: (f, 0, 0)),
            pl.BlockSpec((1, 1, D_MODEL), lambda f, m: (f, 0, 0)),
            pl.BlockSpec((1, 1, D_MODEL), lambda f, m: (f, 0, 0)),
        ],
        out_specs=pl.BlockSpec((1, BM, D_MODEL), lambda f, m: (f, m, 0)),
        out_shape=jax.ShapeDtypeStruct((N_FIELDS, CARD, D_MODEL), jnp.float32),
    )(tables, W, b3, g3, be3)


def _sc_gather_body(xt_hbm, table_hbm, out_hbm, idx_v, rows_v, sem):
    wid = lax.axis_index("s") * NC + lax.axis_index("c")
    base = wid * PER_W
    pltpu.sync_copy(xt_hbm.at[pl.ds(base, PER_W)], idx_v)
    lane = lax.iota(jnp.int32, 16)

    def to_global(k, carry):
        o = k * 16
        f = lax.shift_right_logical(base + o + lane, LOG2_BATCH)
        idx_v[pl.ds(o, 16)] = idx_v[pl.ds(o, 16)] + f * CARD
        return carry

    lax.fori_loop(0, PER_W // 16, to_global, 0)

    def chunk(j, carry):
        pltpu.async_copy(
            table_hbm.at[idx_v.at[pl.ds(j * CH, CH)]], rows_v, sem
        ).wait()
        pltpu.sync_copy(rows_v, out_hbm.at[pl.ds(base + j * CH, CH)])
        return carry

    lax.fori_loop(0, N_CHUNKS, chunk, 0)


@functools.cache
def _make_sc_gather():
    return pl.kernel(
        _sc_gather_body,
        out_type=jax.ShapeDtypeStruct((TOTAL, D_MODEL), jnp.float32),
        mesh=plsc.VectorSubcoreMesh(core_axis_name="c", subcore_axis_name="s"),
        scratch_types=[
            pltpu.VMEM((PER_W,), jnp.int32),
            pltpu.VMEM((CH, D_MODEL), jnp.float32),
            pltpu.SemaphoreType.DMA,
        ],
    )


def kernel(x, tables, W, b, gamma, beta):
    norm_table = _project_tables(tables, W, b, gamma, beta)
    xt = x.T.reshape(-1)
    out = _make_sc_gather()(xt, norm_table.reshape(N_FIELDS * CARD, D_MODEL))
    return out.reshape(N_FIELDS, BATCH, D_MODEL)


# speedup vs baseline: 16.3089x; 1.0201x over previous
import functools

import jax
import jax.numpy as jnp
from jax import lax
from jax.experimental import pallas as pl
from jax.experimental.pallas import tpu as pltpu
from jax.experimental.pallas import tpu_sc as plsc

N_FIELDS = 26
CARD = 10000
EMB_D = 101
D_MODEL = 128
BATCH = 16384
EPS = 1e-5

TOTAL = N_FIELDS * BATCH
BM = 1000

NC = 2
NS = 16
NW = NC * NS
PER_W = TOTAL // NW
CH = 416
N_CHUNKS = PER_W // CH
LOG2_BATCH = 14


def _tc_project_body(tbl_ref, w_ref, b_ref, g_ref, be_ref, out_ref):
    emb = tbl_ref[0]
    w = w_ref[0]
    prj = jnp.dot(emb, w, preferred_element_type=jnp.float32)
    prj = prj + b_ref[0][0][None, :]
    mean = jnp.mean(prj, axis=-1, keepdims=True)
    cent = prj - mean
    var = jnp.mean(cent * cent, axis=-1, keepdims=True)
    inv = lax.rsqrt(var + EPS)
    out_ref[0] = cent * inv * g_ref[0][0][None, :] + be_ref[0][0][None, :]


def _project_tables(tables, W, b, gamma, beta):
    b3 = b[:, None, :]
    g3 = gamma[:, None, :]
    be3 = beta[:, None, :]
    return pl.pallas_call(
        _tc_project_body,
        grid=(N_FIELDS, CARD // BM),
        in_specs=[
            pl.BlockSpec((1, BM, EMB_D), lambda f, m: (f, m, 0)),
            pl.BlockSpec((1, EMB_D, D_MODEL), lambda f, m: (f, 0, 0)),
            pl.BlockSpec((1, 1, D_MODEL), lambda f, m: (f, 0, 0)),
            pl.BlockSpec((1, 1, D_MODEL), lambda f, m: (f, 0, 0)),
            pl.BlockSpec((1, 1, D_MODEL), lambda f, m: (f, 0, 0)),
        ],
        out_specs=pl.BlockSpec((1, BM, D_MODEL), lambda f, m: (f, m, 0)),
        out_shape=jax.ShapeDtypeStruct((N_FIELDS, CARD, D_MODEL), jnp.float32),
    )(tables, W, b3, g3, be3)


def _sc_gather_body(xt_hbm, table_hbm, out_hbm, idx_v, rows_v, g_sem, w_sem):
    wid = lax.axis_index("s") * NC + lax.axis_index("c")
    base = wid * PER_W
    pltpu.sync_copy(xt_hbm.at[pl.ds(base, PER_W)], idx_v)
    lane = lax.iota(jnp.int32, 16)

    def to_global(k, carry):
        o = k * 16
        f = lax.shift_right_logical(base + o + lane, LOG2_BATCH)
        idx_v[pl.ds(o, 16)] = idx_v[pl.ds(o, 16)] + f * CARD
        return carry

    lax.fori_loop(0, PER_W // 16, to_global, 0)

    def gather(j, buf):
        return pltpu.make_async_copy(
            table_hbm.at[idx_v.at[pl.ds(j * CH, CH)]], rows_v.at[buf], g_sem
        )

    def write(j, buf):
        return pltpu.make_async_copy(
            rows_v.at[buf], out_hbm.at[pl.ds(base + j * CH, CH)], w_sem
        )

    gather(0, 0).start()
    gather(0, 0).wait()
    write(0, 0).start()
    gather(1, 1).start()

    def chunk(j, carry):
        b = lax.rem(j, 2)
        gather(j, b).wait()
        write(j, b).start()
        write(j - 1, 1 - b).wait()
        gather(j + 1, 1 - b).start()
        return carry

    lax.fori_loop(1, N_CHUNKS - 1, chunk, 0)

    last = N_CHUNKS - 1
    lb = last % 2
    gather(last, lb).wait()
    write(last, lb).start()
    write(last - 1, 1 - lb).wait()
    write(last, lb).wait()


@functools.cache
def _make_sc_gather():
    return pl.kernel(
        _sc_gather_body,
        out_type=jax.ShapeDtypeStruct((TOTAL, D_MODEL), jnp.float32),
        mesh=plsc.VectorSubcoreMesh(core_axis_name="c", subcore_axis_name="s"),
        scratch_types=[
            pltpu.VMEM((PER_W,), jnp.int32),
            pltpu.VMEM((2, CH, D_MODEL), jnp.float32),
            pltpu.SemaphoreType.DMA,
            pltpu.SemaphoreType.DMA,
        ],
    )


def kernel(x, tables, W, b, gamma, beta):
    norm_table = _project_tables(tables, W, b, gamma, beta)
    xt = x.T.reshape(-1)
    out = _make_sc_gather()(xt, norm_table.reshape(N_FIELDS * CARD, D_MODEL))
    return out.reshape(N_FIELDS, BATCH, D_MODEL)
